# baseline (speedup 1.0000x reference)
import functools

import jax
import jax.numpy as jnp
from jax import lax
from jax.experimental import pallas as pl
from jax.experimental.pallas import tpu as pltpu
from jax.experimental.pallas import tpu_sc as plsc

_NU, _NP, _NM, _NT = 5000, 20000, 2000, 1000
_D, _DH = 768, 1024
_H1, _C1 = 8, 128
_K = 256
_NSC, _NTILE = 2, 16
_NW = _NSC * _NTILE

_TYPES = ("user", "paper", "method", "task")
_NNODE = {"user": _NU, "paper": _NP, "method": _NM, "task": _NT}
_R_SRC = ("paper", "paper", "method", "paper", "task", "user", "paper")
_R_DST = ("paper", "method", "paper", "task", "paper", "paper", "user")
_E_KEY = {"cites": 20000, "applies": 10000, "performs": 10000, "likes": 12000}

_L1_JOBS = [(0, "cites", False), (0, "cites", True), (1, "applies", False),
            (2, "applies", True), (3, "performs", False), (4, "performs", True),
            (5, "likes", False), (6, "likes", True)]
_L2_JOBS = [(0, "cites", False), (0, "cites", True), (2, "applies", True),
            (4, "performs", True), (5, "likes", False), (6, "likes", True)]

_L1_RELS = (0, 1, 2, 3, 4, 5, 6)
_L2_RELS = (0, 2, 4, 5, 6)


def _cdiv(a, b):
    return (a + b - 1) // b


def _epad(e):
    return _cdiv(e, _K) * _K


def _npad(n):
    return _cdiv(n, 1024) * 1024


def _den_layout(rels):
    bases, off = {}, 0
    for r in rels:
        bases[r] = off
        off += _NNODE[_R_DST[r]]
    off = _cdiv(off, 2048) * 2048
    return bases, off


def _alpha_layout(jobs):
    bases, off = [], 0
    for (_r, key, _s) in jobs:
        bases.append(off)
        off += _epad(_E_KEY[key])
    return bases, off


def _mm_body(relu, low, cin, cout, cb, odt, x_ref, w_ref, o_ref):
    a = x_ref[...]
    if cin:
        a = a.transpose(1, 0, 2).reshape(a.shape[1], a.shape[0] * 64)
    if relu:
        a = jnp.maximum(a, 0.0)
    b = w_ref[...]
    if low:
        a = a.astype(jnp.bfloat16)
        b = b.astype(jnp.bfloat16)
    else:
        a = a.astype(jnp.float32)
    res = jnp.dot(a, b, preferred_element_type=jnp.float32)
    res = res.astype(odt)
    if cout:
        res = res.reshape(res.shape[0], cb, 64).transpose(1, 0, 2)
    o_ref[...] = res


def _matmul(x, w, relu=False, low=False, cin=False, cout=False, bm=256,
            odt=jnp.float32):
    if cin:
        nch_in, m, _ = x.shape
        k = nch_in * 64
    else:
        m, k = x.shape
    n = w.shape[1]
    bn = 512 if n % 512 == 0 else 256
    cb = bn // 64
    grid = (_cdiv(m, bm), _cdiv(n, bn))
    if cin:
        x_spec = pl.BlockSpec((nch_in, bm, 64), lambda i, j: (0, i, 0))
    else:
        x_spec = pl.BlockSpec((bm, k), lambda i, j: (i, 0))
    if cout:
        o_spec = pl.BlockSpec((cb, bm, 64), lambda i, j: (j, i, 0))
        o_shape = jax.ShapeDtypeStruct((n // 64, m, 64), odt)
    else:
        o_spec = pl.BlockSpec((bm, bn), lambda i, j: (i, j))
        o_shape = jax.ShapeDtypeStruct((m, n), odt)
    return pl.pallas_call(
        functools.partial(_mm_body, relu, low, cin, cout, cb, odt),
        grid=grid,
        in_specs=[x_spec, pl.BlockSpec((k, bn), lambda i, j: (0, j))],
        out_specs=o_spec,
        out_shape=o_shape,
    )(x, w)


def _fold_body(h, c, w_ref, as_ref, ad_ref, o_ref):
    d = w_ref.shape[1]
    w = w_ref[0].reshape(d, h, c)
    was = (w * as_ref[0][None]).sum(-1)
    wad = (w * ad_ref[0][None]).sum(-1)
    z = jnp.zeros((d, 16 - h), jnp.float32)
    o_ref[0] = jnp.concatenate([was, z, wad, z], axis=1)


def _fold(w, a_s, a_d, h, c):
    r, d, _ = w.shape
    out = pl.pallas_call(
        functools.partial(_fold_body, h, c),
        grid=(r,),
        in_specs=[pl.BlockSpec((1, d, h * c), lambda i: (i, 0, 0)),
                  pl.BlockSpec((1, h, c), lambda i: (i, 0, 0)),
                  pl.BlockSpec((1, h, c), lambda i: (i, 0, 0))],
        out_specs=pl.BlockSpec((1, d, 32), lambda i: (i, 0, 0)),
        out_shape=jax.ShapeDtypeStruct((r, d, 32), jnp.float32),
    )(w, a_s, a_d)
    return out.transpose(1, 0, 2).reshape(d, r * 32)


_SC_PARAMS = pltpu.CompilerParams(use_tc_tiling_on_sc=False,
                                  needs_layout_passes=False)


def _mesh():
    return plsc.VectorSubcoreMesh(core_axis_name="c", subcore_axis_name="s",
                                  num_cores=_NSC, num_subcores=_NTILE)


def _job_edges(e_refs, key, swap):
    s_ref, d_ref = e_refs[key]
    return (d_ref, s_ref) if swap else (s_ref, d_ref)


def _edge_keys(jobs):
    seen = []
    for (_r, key, _s) in jobs:
        if key not in seen:
            seen.append(key)
    return seen


def _den_pass(jobs, den_bases, den_n, as_tabs, ad_tabs, edge_pads):
    keys = _edge_keys(jobs)
    n_e = 2 * len(keys)
    n_j = len(jobs)
    rows_t = den_n // _NTILE

    def body(*refs):
        e_flat = refs[:n_e]
        asr = refs[n_e:n_e + n_j]
        adr = refs[n_e + n_j:n_e + 2 * n_j]
        den0_o, den1_o = refs[n_e + 2 * n_j], refs[n_e + 2 * n_j + 1]
        (acc, idx_s, idx_d, idx_p, asb, adb, exb, zb) = refs[n_e + 2 * n_j + 2:]
        e_refs = {k: (e_flat[2 * i], e_flat[2 * i + 1])
                  for i, k in enumerate(keys)}
        c = lax.axis_index("c")
        s = lax.axis_index("s")
        wid = c * _NTILE + s

        zbr = rows_t // 16

        def zrow(m, carry):
            zb[m] = jnp.zeros((16,), jnp.float32)
            return carry
        lax.fori_loop(0, zbr, zrow, None)
        for q in range(16):
            pltpu.sync_copy(zb, acc.at[pl.ds(s * rows_t + q * zbr, zbr)])
        plsc.subcore_barrier()

        for ji, (r, key, swap) in enumerate(jobs):
            s_ref, d_ref = _job_edges(e_refs, key, swap)
            e_cnt = _E_KEY[key]
            nb = _epad(e_cnt) // _K

            def blk(i, carry, s_ref=s_ref, d_ref=d_ref, a_tab=asr[ji],
                    d_tab=adr[ji], dbase=den_bases[r], e_cnt=e_cnt):
                base = (wid + i * _NW) * _K
                pltpu.sync_copy(s_ref.at[pl.ds(base, _K)], idx_s)
                pltpu.sync_copy(d_ref.at[pl.ds(base, _K)], idx_d)
                pltpu.sync_copy(a_tab.at[idx_s], asb)
                pltpu.sync_copy(d_tab.at[idx_d], adb)

                def col(m, cc):
                    idx_p[pl.ds(m * 16, 16)] = idx_d[pl.ds(m * 16, 16)] + dbase
                    return cc
                lax.fori_loop(0, _K // 16, col, None, unroll=4)

                def edge(k, cc):
                    e = asb[k] + adb[k]
                    e = jnp.where(e > 0, e, 0.2 * e)
                    ex = jnp.exp(jnp.minimum(e, 60.0))
                    valid = jnp.where(base + k < e_cnt, 1.0, 0.0)
                    exb[k] = ex * valid
                    return cc
                lax.fori_loop(0, _K, edge, None, unroll=4)
                pltpu.sync_copy(exb, acc.at[idx_p], add=True)
                return carry

            nblk = (nb - wid + _NW - 1) // _NW
            lax.fori_loop(0, nblk, blk, None)

        plsc.subcore_barrier()
        sl = pl.ds(s * rows_t, rows_t)

        @pl.when(c == 0)
        def _():
            pltpu.sync_copy(acc.at[sl], den0_o.at[sl])

        @pl.when(c == 1)
        def _():
            pltpu.sync_copy(acc.at[sl], den1_o.at[sl])

    out_t = (jax.ShapeDtypeStruct((den_n, 16), jnp.float32),
             jax.ShapeDtypeStruct((den_n, 16), jnp.float32))
    scratch = [
        pltpu.VMEM_SHARED((den_n, 16), jnp.float32),
        pltpu.VMEM((_K,), jnp.int32),
        pltpu.VMEM((_K,), jnp.int32),
        pltpu.VMEM((_K,), jnp.int32),
        pltpu.VMEM((_K, 16), jnp.float32),
        pltpu.VMEM((_K, 16), jnp.float32),
        pltpu.VMEM((_K, 16), jnp.float32),
        pltpu.VMEM((rows_t // 16, 16), jnp.float32),
    ]
    args = []
    for k in keys:
        args.extend(edge_pads[k])
    args.extend(as_tabs)
    args.extend(ad_tabs)
    kern = pl.kernel(body, out_type=out_t, mesh=_mesh(), scratch_types=scratch,
                     compiler_params=_SC_PARAMS)
    return kern(*args)


def _alpha_pass(jobs, den_bases, a_bases, a_tot, as_tabs, ad_tabs,
                den0, den1, edge_pads):
    keys = _edge_keys(jobs)
    n_e = 2 * len(keys)
    n_j = len(jobs)

    def body(*refs):
        e_flat = refs[:n_e]
        asr = refs[n_e:n_e + n_j]
        adr = refs[n_e + n_j:n_e + 2 * n_j]
        den0_r, den1_r = refs[n_e + 2 * n_j], refs[n_e + 2 * n_j + 1]
        alpha_o = refs[n_e + 2 * n_j + 2]
        (idx_s, idx_d, idx_p, asb, adb, d0b, d1b, ab) = refs[n_e + 2 * n_j + 3:]
        e_refs = {k: (e_flat[2 * i], e_flat[2 * i + 1])
                  for i, k in enumerate(keys)}
        c = lax.axis_index("c")
        s = lax.axis_index("s")
        wid = c * _NTILE + s

        for ji, (r, key, swap) in enumerate(jobs):
            s_ref, d_ref = _job_edges(e_refs, key, swap)
            e_cnt = _E_KEY[key]
            nb = _epad(e_cnt) // _K

            def blk(i, carry, s_ref=s_ref, d_ref=d_ref, a_tab=asr[ji],
                    d_tab=adr[ji], dbase=den_bases[r], e_cnt=e_cnt,
                    abase=a_bases[ji]):
                base = (wid + i * _NW) * _K
                pltpu.sync_copy(s_ref.at[pl.ds(base, _K)], idx_s)
                pltpu.sync_copy(d_ref.at[pl.ds(base, _K)], idx_d)
                pltpu.sync_copy(a_tab.at[idx_s], asb)
                pltpu.sync_copy(d_tab.at[idx_d], adb)

                def col(m, cc):
                    idx_p[pl.ds(m * 16, 16)] = idx_d[pl.ds(m * 16, 16)] + dbase
                    return cc
                lax.fori_loop(0, _K // 16, col, None, unroll=4)
                pltpu.sync_copy(den0_r.at[idx_p], d0b)
                pltpu.sync_copy(den1_r.at[idx_p], d1b)

                def edge(k, cc):
                    e = asb[k] + adb[k]
                    e = jnp.where(e > 0, e, 0.2 * e)
                    ex = jnp.exp(jnp.minimum(e, 60.0))
                    valid = jnp.where(base + k < e_cnt, 1.0, 0.0)
                    den = d0b[k] + d1b[k] + 1e-16
                    ab[k] = ex * valid / den
                    return cc
                lax.fori_loop(0, _K, edge, None, unroll=4)
                pltpu.sync_copy(ab, alpha_o.at[pl.ds(abase + base, _K)])
                return carry

            nblk = (nb - wid + _NW - 1) // _NW
            lax.fori_loop(0, nblk, blk, None)

    out_t = jax.ShapeDtypeStruct((a_tot, 16), jnp.float32)
    scratch = ([pltpu.VMEM((_K,), jnp.int32)] * 3 +
               [pltpu.VMEM((_K, 16), jnp.float32)] * 5)
    args = []
    for k in keys:
        args.extend(edge_pads[k])
    args.extend(as_tabs)
    args.extend(ad_tabs)
    args.extend([den0, den1])
    kern = pl.kernel(body, out_type=out_t, mesh=_mesh(), scratch_types=scratch,
                     compiler_params=_SC_PARAMS)
    return kern(*args)


def _msg_pass(jobs, a_bases, alpha, tabs, tab_of_r, groups, nch, heads,
              edge_pads, biases, pdt):
    keys = _edge_keys(jobs)
    n_e = 2 * len(keys)
    n_x = len(_TYPES)
    n_g = len(groups)
    acc_rows = max(_npad(_NNODE[g[0]]) for g in groups)

    def body(*refs):
        e_flat = refs[:n_e]
        xst = {t: refs[n_e + i] for i, t in enumerate(_TYPES)}
        alpha_r = refs[n_e + n_x]
        b_refs = refs[n_e + n_x + 1:n_e + n_x + 1 + n_g]
        outs = refs[n_e + n_x + 1 + n_g:n_e + n_x + 1 + 2 * n_g]
        (acc, idx_s, idx_d, arows, rows, zb, bbuf, sem0, sem1) = \
            refs[n_e + n_x + 1 + 2 * n_g:]
        e_refs = {k: (e_flat[2 * i], e_flat[2 * i + 1])
                  for i, k in enumerate(keys)}
        c = lax.axis_index("c")
        s = lax.axis_index("s")

        for gi, (dst_t, job_ids) in enumerate(groups):
            npad_d = _npad(_NNODE[dst_t])
            rt = npad_d // _NTILE

            def chunk(cc_l, carry, gi=gi, job_ids=job_ids, rt=rt):
                cc = cc_l * 2 + c
                lane = cc_l if heads > 1 else 0
                pltpu.sync_copy(b_refs[gi].at[pl.ds(cc * 64, 64)], bbuf)
                lanes = 32 if pdt == jnp.bfloat16 else 16
                nq = 64 // lanes

                def brow(m, bc):
                    q = m % nq
                    sl = pl.ds(q * lanes, lanes)
                    zb[m // nq, sl] = bbuf[sl]
                    return bc
                lax.fori_loop(0, 64 * nq, brow, None)
                for q in range(rt // 64):
                    pltpu.sync_copy(zb, acc.at[pl.ds(s * rt + q * 64, 64)])
                plsc.subcore_barrier()

                for ji in job_ids:
                    r, key, swap = jobs[ji]
                    s_ref, d_ref = _job_edges(e_refs, key, swap)
                    nb = _epad(_E_KEY[key]) // _K
                    nblk = (nb - s + _NTILE - 1) // _NTILE
                    t_r, cbase = tab_of_r[r]
                    tab = xst[t_r]
                    tcc = cbase + cc
                    abase = a_bases[ji]
                    sems = (sem0, sem1)

                    def fetch(bi, slot, s_ref=s_ref, d_ref=d_ref, tab=tab,
                              abase=abase, cc=tcc, sems=sems):
                        base = (s + bi * _NTILE) * _K
                        pltpu.sync_copy(s_ref.at[pl.ds(base, _K)],
                                        idx_s.at[slot])
                        pltpu.sync_copy(d_ref.at[pl.ds(base, _K)],
                                        idx_d.at[slot])
                        pltpu.sync_copy(alpha_r.at[pl.ds(abase + base, _K)],
                                        arows.at[slot])
                        pltpu.async_copy(tab.at[cc].at[idx_s.at[slot]],
                                         rows.at[slot], sems[slot])

                    @pl.when(nblk > 0)
                    def _(fetch=fetch):
                        fetch(0, 0)

                    def pair(p, bc, fetch=fetch, tab=tab, cc=tcc, lane=lane,
                             nblk=nblk, sems=sems):
                        ll = jnp.full((16,), lane, jnp.int32)
                        for b in (0, 1):
                            bi = 2 * p + b

                            @pl.when(bi < nblk)
                            def _(bi=bi, b=b):
                                pltpu.make_async_copy(
                                    tab.at[cc].at[idx_s.at[b]],
                                    rows.at[b], sems[b]).wait()

                                @pl.when(bi + 1 < nblk)
                                def _():
                                    fetch(bi + 1, 1 - b)

                                def edge(k, ec):
                                    kk = jnp.full((16,), k, jnp.int32)
                                    av = plsc.load_gather(arows.at[b],
                                                          [kk, ll])
                                    if pdt == jnp.bfloat16:
                                        for q in range(2):
                                            sl = pl.ds(q * 32, 32)
                                            v = rows[b, k, sl]
                                            lo, hi = plsc.unpack(
                                                v, format=plsc.PackFormat
                                                .INTERLEAVED)
                                            rows[b, k, sl] = plsc.pack(
                                                lo * av, hi * av,
                                                format=plsc.PackFormat
                                                .INTERLEAVED)
                                    else:
                                        for q in range(4):
                                            sl = pl.ds(q * 16, 16)
                                            rows[b, k, sl] = \
                                                rows[b, k, sl] * av
                                    return ec
                                lax.fori_loop(0, _K, edge, None, unroll=4)
                                pltpu.sync_copy(rows.at[b],
                                                acc.at[idx_d.at[b]], add=True)
                        return bc

                    lax.fori_loop(0, (nblk + 1) // 2, pair, None)

                plsc.subcore_barrier()
                sl = pl.ds(s * rt, rt)
                pltpu.sync_copy(acc.at[sl], outs[gi].at[cc].at[sl])
                plsc.subcore_barrier()
                return carry

            lax.fori_loop(0, nch // 2, chunk, None)

    out_t = tuple(jax.ShapeDtypeStruct((nch, _npad(_NNODE[g[0]]), 64),
                                       pdt) for g in groups)
    scratch = [
        pltpu.VMEM_SHARED((acc_rows, 64), pdt),
        pltpu.VMEM((2, _K), jnp.int32),
        pltpu.VMEM((2, _K), jnp.int32),
        pltpu.VMEM((2, _K, 16), jnp.float32),
        pltpu.VMEM((2, _K, 64), pdt),
        pltpu.VMEM((64, 64), pdt),
        pltpu.VMEM((64,), pdt),
        pltpu.SemaphoreType.DMA,
        pltpu.SemaphoreType.DMA,
    ]
    args = []
    for k in keys:
        args.extend(edge_pads[k])
    args.extend(tabs[t] for t in _TYPES)
    args.append(alpha)
    args.extend(biases)
    kern = pl.kernel(body, out_type=out_t, mesh=_mesh(), scratch_types=scratch,
                     compiler_params=_SC_PARAMS)
    return kern(*args)


_KF = 64


def _edge_dot(eli_u, eli_p, hu, hp, e_pad):
    d = hu.shape[1]
    nq = d // 16

    def body(u_ref, p_ref, hu_ref, hp_ref, out_ref,
             idx_u, idx_p, urows, prows, resb):
        c = lax.axis_index("c")
        s = lax.axis_index("s")
        wid = c * _NTILE + s
        nb = e_pad // _KF

        def blk(i, carry):
            base = (wid + i * _NW) * _KF
            pltpu.sync_copy(u_ref.at[pl.ds(base, _KF)], idx_u)
            pltpu.sync_copy(p_ref.at[pl.ds(base, _KF)], idx_p)
            pltpu.sync_copy(hu_ref.at[idx_u], urows)
            pltpu.sync_copy(hp_ref.at[idx_p], prows)

            iot = lax.iota(jnp.int32, 16)

            def egrp(g, ec):
                kk = iot + g * 16

                def colj(j, acc):
                    jj = jnp.full((16,), j, jnp.int32)
                    uj = plsc.load_gather(urows, [kk, jj])
                    pj = plsc.load_gather(prows, [kk, jj])
                    return acc + uj * pj
                res = lax.fori_loop(0, d, colj, jnp.zeros((16,), jnp.float32), unroll=8)
                resb[pl.ds(g * 16, 16)] = res
                return ec
            lax.fori_loop(0, _KF // 16, egrp, None)
            pltpu.sync_copy(resb, out_ref.at[pl.ds(base, _KF)])
            return carry

        nblk = (nb - wid + _NW - 1) // _NW
        lax.fori_loop(0, nblk, blk, None)

    out_t = jax.ShapeDtypeStruct((e_pad,), jnp.float32)
    scratch = [
        pltpu.VMEM((_KF,), jnp.int32),
        pltpu.VMEM((_KF,), jnp.int32),
        pltpu.VMEM((_KF, d), jnp.float32),
        pltpu.VMEM((_KF, d), jnp.float32),
        pltpu.VMEM((_KF,), jnp.float32),
    ]
    kern = pl.kernel(body, out_type=out_t, mesh=_mesh(), scratch_types=scratch,
                     compiler_params=_SC_PARAMS)
    return kern(eli_u, eli_p, hu, hp)


def _to_chunks(y):
    n, w = y.shape
    return y.reshape(n, w // 64, 64).transpose(1, 0, 2)


def _from_chunks(yc, n):
    return yc.transpose(1, 0, 2).reshape(yc.shape[1], -1)[:n]


def _pad1(a, n):
    a = a.astype(jnp.int32)
    return jnp.concatenate([a, jnp.zeros((n - a.shape[0],), jnp.int32)])


def _layer(xs_by_type, jobs, rels, big, w_cat, wf, relu, cin, edge_pads, nch,
           heads, groups, biases, pdt):
    den_bases, den_n = _den_layout(rels)
    a_bases, a_tot = _alpha_layout(jobs)

    yb = {t: _matmul(xs_by_type[t], w_cat[t], relu=relu, low=True,
                     cin=cin, cout=True, odt=pdt) for t in _TYPES}
    ya = {t: _matmul(xs_by_type[t], wf, relu=relu, cin=cin) for t in _TYPES}

    tab_of_r, as_by_r, ad_by_r = {}, {}, {}
    for t in _TYPES:
        for i, r in enumerate(big[t]):
            tab_of_r[r] = (t, i * nch)
        for r in rels:
            if _R_SRC[r] == t:
                as_by_r[r] = ya[t][:, r * 32: r * 32 + 16]
            if _R_DST[r] == t:
                ad_by_r[r] = ya[t][:, r * 32 + 16: r * 32 + 32]

    as_list = tuple(as_by_r[r] for r, _k, _s in jobs)
    ad_list = tuple(ad_by_r[r] for r, _k, _s in jobs)
    den0, den1 = _den_pass(jobs, den_bases, den_n, as_list, ad_list, edge_pads)
    alpha = _alpha_pass(jobs, den_bases, a_bases, a_tot, as_list, ad_list,
                        den0, den1, edge_pads)
    outs = _msg_pass(jobs, a_bases, alpha, yb, tab_of_r, groups, nch, heads,
                     edge_pads, [b.astype(pdt) for b in biases], pdt)
    return {g[0]: o for g, o in zip(groups, outs)}


def kernel(x_user, x_paper, x_method, x_task, ei_cites, ei_applies,
           ei_performs, ei_likes, edge_label_index, W1, as1, ad1, b1,
           W2, as2, ad2, b2):
    xs = {"user": x_user, "paper": x_paper, "method": x_method,
          "task": x_task}

    edge_pads = {}
    for key, ei in (("cites", ei_cites), ("applies", ei_applies),
                    ("performs", ei_performs), ("likes", ei_likes)):
        ep = _epad(_E_KEY[key])
        edge_pads[key] = (_pad1(ei[0], ep), _pad1(ei[1], ep))

    wf1 = _fold(W1, as1, ad1, _H1, _C1)
    wf2 = _fold(W2, as2, ad2, 1, _D)
    wf1 = jnp.concatenate([wf1, jnp.zeros((_D, 32), jnp.float32)], axis=1)
    wf2 = jnp.concatenate([wf2, jnp.zeros((_DH, 32), jnp.float32)], axis=1)

    m_dst = jnp.array([[1.0 if _R_DST[r] == t else 0.0 for r in range(7)]
                       for t in _TYPES], jnp.float32)
    b1s = _matmul(m_dst, b1, bm=8)
    b2s = _matmul(m_dst, b2, bm=8)

    big1 = {"paper": [0, 1, 3, 6], "method": [2], "task": [4], "user": [5]}
    w_cat1 = {t: jnp.concatenate([W1[r] for r in big1[t]], axis=1)
              for t in _TYPES}
    groups1 = [("paper", [0, 1, 3, 5, 6]), ("method", [2]), ("task", [4]),
               ("user", [7])]
    bias_g1 = [b1s[1], b1s[2], b1s[3], b1s[0]]
    outs1 = _layer(xs, _L1_JOBS, _L1_RELS, big1, w_cat1, wf1, False, False,
                   edge_pads, 16, _H1, groups1, bias_g1, jnp.bfloat16)

    big2 = {"paper": [0, 6], "method": [2], "task": [4], "user": [5]}
    w_cat2 = {t: jnp.concatenate([W2[r] for r in big2[t]], axis=1)
              for t in _TYPES}
    groups2 = [("paper", [0, 1, 2, 3, 4]), ("user", [5])]
    bias_g2 = [b2s[1], b2s[0]]
    outs2 = _layer(outs1, _L2_JOBS, _L2_RELS, big2, w_cat2, wf2, True, True,
                   edge_pads, 12, 1, groups2, bias_g2, jnp.float32)
    h2u = _from_chunks(outs2["user"], _NU).astype(jnp.float32)
    h2p = _from_chunks(outs2["paper"], _NP).astype(jnp.float32)

    el = edge_label_index.shape[1]
    el_pad = _cdiv(el, _KF * _NW) * _KF * _NW
    eli_u = _pad1(edge_label_index[0], el_pad)
    eli_p = _pad1(edge_label_index[1], el_pad)
    res = _edge_dot(eli_u, eli_p, h2u, h2p, el_pad)
    return res[:el]


# speedup vs baseline: 1.0178x; 1.0178x over previous
import functools

import jax
import jax.numpy as jnp
from jax import lax
from jax.experimental import pallas as pl
from jax.experimental.pallas import tpu as pltpu
from jax.experimental.pallas import tpu_sc as plsc

_NU, _NP, _NM, _NT = 5000, 20000, 2000, 1000
_D, _DH = 768, 1024
_H1, _C1 = 8, 128
_K = 256
_NSC, _NTILE = 2, 16
_NW = _NSC * _NTILE

_TYPES = ("user", "paper", "method", "task")
_NNODE = {"user": _NU, "paper": _NP, "method": _NM, "task": _NT}
_R_SRC = ("paper", "paper", "method", "paper", "task", "user", "paper")
_R_DST = ("paper", "method", "paper", "task", "paper", "paper", "user")
_E_KEY = {"cites": 20000, "applies": 10000, "performs": 10000, "likes": 12000}

_L1_JOBS = [(0, "cites", False), (0, "cites", True), (1, "applies", False),
            (2, "applies", True), (3, "performs", False), (4, "performs", True),
            (5, "likes", False), (6, "likes", True)]
_L2_JOBS = [(0, "cites", False), (0, "cites", True), (2, "applies", True),
            (4, "performs", True), (5, "likes", False), (6, "likes", True)]

_L1_RELS = (0, 1, 2, 3, 4, 5, 6)
_L2_RELS = (0, 2, 4, 5, 6)


def _cdiv(a, b):
    return (a + b - 1) // b


def _epad(e):
    return _cdiv(e, _K) * _K


def _npad(n):
    return _cdiv(n, 1024) * 1024


def _den_layout(rels):
    bases, off = {}, 0
    for r in rels:
        bases[r] = off
        off += _NNODE[_R_DST[r]]
    off = _cdiv(off, 2048) * 2048
    return bases, off


def _alpha_layout(jobs):
    bases, off = [], 0
    for (_r, key, _s) in jobs:
        bases.append(off)
        off += _epad(_E_KEY[key])
    return bases, off


def _mm_body(relu, low, cin, cout, cb, odt, x_ref, w_ref, o_ref):
    a = x_ref[...]
    if cin:
        a = a.transpose(1, 0, 2).reshape(a.shape[1], a.shape[0] * 64)
    if relu:
        a = jnp.maximum(a, 0.0)
    b = w_ref[...]
    if low:
        a = a.astype(jnp.bfloat16)
        b = b.astype(jnp.bfloat16)
    else:
        a = a.astype(jnp.float32)
    res = jnp.dot(a, b, preferred_element_type=jnp.float32)
    res = res.astype(odt)
    if cout:
        res = res.reshape(res.shape[0], cb, 64).transpose(1, 0, 2)
    o_ref[...] = res


def _matmul(x, w, relu=False, low=False, cin=False, cout=False, bm=256,
            odt=jnp.float32):
    if cin:
        nch_in, m, _ = x.shape
        k = nch_in * 64
    else:
        m, k = x.shape
    n = w.shape[1]
    bn = 512 if n % 512 == 0 else 256
    cb = bn // 64
    grid = (_cdiv(m, bm), _cdiv(n, bn))
    if cin:
        x_spec = pl.BlockSpec((nch_in, bm, 64), lambda i, j: (0, i, 0))
    else:
        x_spec = pl.BlockSpec((bm, k), lambda i, j: (i, 0))
    if cout:
        o_spec = pl.BlockSpec((cb, bm, 64), lambda i, j: (j, i, 0))
        o_shape = jax.ShapeDtypeStruct((n // 64, m, 64), odt)
    else:
        o_spec = pl.BlockSpec((bm, bn), lambda i, j: (i, j))
        o_shape = jax.ShapeDtypeStruct((m, n), odt)
    return pl.pallas_call(
        functools.partial(_mm_body, relu, low, cin, cout, cb, odt),
        grid=grid,
        in_specs=[x_spec, pl.BlockSpec((k, bn), lambda i, j: (0, j))],
        out_specs=o_spec,
        out_shape=o_shape,
    )(x, w)


def _fold_body(h, c, w_ref, as_ref, ad_ref, o_ref):
    d = w_ref.shape[1]
    w = w_ref[0].reshape(d, h, c)
    was = (w * as_ref[0][None]).sum(-1)
    wad = (w * ad_ref[0][None]).sum(-1)
    z = jnp.zeros((d, 16 - h), jnp.float32)
    o_ref[0] = jnp.concatenate([was, z, wad, z], axis=1)


def _fold(w, a_s, a_d, h, c):
    r, d, _ = w.shape
    out = pl.pallas_call(
        functools.partial(_fold_body, h, c),
        grid=(r,),
        in_specs=[pl.BlockSpec((1, d, h * c), lambda i: (i, 0, 0)),
                  pl.BlockSpec((1, h, c), lambda i: (i, 0, 0)),
                  pl.BlockSpec((1, h, c), lambda i: (i, 0, 0))],
        out_specs=pl.BlockSpec((1, d, 32), lambda i: (i, 0, 0)),
        out_shape=jax.ShapeDtypeStruct((r, d, 32), jnp.float32),
    )(w, a_s, a_d)
    return out.transpose(1, 0, 2).reshape(d, r * 32)


_SC_PARAMS = pltpu.CompilerParams(use_tc_tiling_on_sc=False,
                                  needs_layout_passes=False)


def _mesh():
    return plsc.VectorSubcoreMesh(core_axis_name="c", subcore_axis_name="s",
                                  num_cores=_NSC, num_subcores=_NTILE)


def _job_edges(e_refs, key, swap):
    s_ref, d_ref = e_refs[key]
    return (d_ref, s_ref) if swap else (s_ref, d_ref)


def _edge_keys(jobs):
    seen = []
    for (_r, key, _s) in jobs:
        if key not in seen:
            seen.append(key)
    return seen


def _den_pass(jobs, den_bases, den_n, as_tabs, ad_tabs, edge_pads):
    keys = _edge_keys(jobs)
    n_e = 2 * len(keys)
    n_j = len(jobs)
    rows_t = den_n // _NTILE

    def body(*refs):
        e_flat = refs[:n_e]
        asr = refs[n_e:n_e + n_j]
        adr = refs[n_e + n_j:n_e + 2 * n_j]
        den0_o, den1_o = refs[n_e + 2 * n_j], refs[n_e + 2 * n_j + 1]
        (acc, idx_s, idx_d, idx_p, asb, adb, exb, zb, sem0, sem1) = \
            refs[n_e + 2 * n_j + 2:]
        e_refs = {k: (e_flat[2 * i], e_flat[2 * i + 1])
                  for i, k in enumerate(keys)}
        c = lax.axis_index("c")
        s = lax.axis_index("s")
        wid = c * _NTILE + s

        zbr = rows_t // 16

        def zrow(m, carry):
            zb[m] = jnp.zeros((16,), jnp.float32)
            return carry
        lax.fori_loop(0, zbr, zrow, None)
        for q in range(16):
            pltpu.sync_copy(zb, acc.at[pl.ds(s * rows_t + q * zbr, zbr)])
        plsc.subcore_barrier()

        for ji, (r, key, swap) in enumerate(jobs):
            s_ref, d_ref = _job_edges(e_refs, key, swap)
            e_cnt = _E_KEY[key]
            nb = _epad(e_cnt) // _K
            nblk = (nb - wid + _NW - 1) // _NW
            a_tab, d_tab = asr[ji], adr[ji]
            dbase = den_bases[r]
            sems = (sem0, sem1)

            def fetch(bi, slot, s_ref=s_ref, d_ref=d_ref, a_tab=a_tab,
                      d_tab=d_tab, sems=sems):
                base = (wid + bi * _NW) * _K
                pltpu.sync_copy(s_ref.at[pl.ds(base, _K)], idx_s.at[slot])
                pltpu.sync_copy(d_ref.at[pl.ds(base, _K)], idx_d.at[slot])
                pltpu.async_copy(a_tab.at[idx_s.at[slot]], asb.at[slot],
                                 sems[slot])
                pltpu.async_copy(d_tab.at[idx_d.at[slot]], adb.at[slot],
                                 sems[slot])

            @pl.when(nblk > 0)
            def _(fetch=fetch):
                fetch(0, 0)

            def pair(p, carry, fetch=fetch, a_tab=a_tab, d_tab=d_tab,
                     dbase=dbase, e_cnt=e_cnt, nblk=nblk, sems=sems):
                for b in (0, 1):
                    bi = 2 * p + b

                    @pl.when(bi < nblk)
                    def _(bi=bi, b=b):
                        pltpu.make_async_copy(a_tab.at[idx_s.at[b]],
                                              asb.at[b], sems[b]).wait()
                        pltpu.make_async_copy(d_tab.at[idx_d.at[b]],
                                              adb.at[b], sems[b]).wait()

                        @pl.when(bi + 1 < nblk)
                        def _():
                            fetch(bi + 1, 1 - b)
                        base = (wid + bi * _NW) * _K

                        def col(m, cc):
                            idx_p[pl.ds(m * 16, 16)] = \
                                idx_d[b, pl.ds(m * 16, 16)] + dbase
                            return cc
                        lax.fori_loop(0, _K // 16, col, None, unroll=4)

                        def edge(k, cc):
                            e = asb[b, k] + adb[b, k]
                            e = jnp.where(e > 0, e, 0.2 * e)
                            ex = jnp.exp(jnp.minimum(e, 60.0))
                            valid = jnp.where(base + k < e_cnt, 1.0, 0.0)
                            exb[k] = ex * valid
                            return cc
                        lax.fori_loop(0, _K, edge, None, unroll=4)
                        pltpu.sync_copy(exb, acc.at[idx_p], add=True)
                return carry

            lax.fori_loop(0, (nblk + 1) // 2, pair, None)

        plsc.subcore_barrier()
        sl = pl.ds(s * rows_t, rows_t)

        @pl.when(c == 0)
        def _():
            pltpu.sync_copy(acc.at[sl], den0_o.at[sl])

        @pl.when(c == 1)
        def _():
            pltpu.sync_copy(acc.at[sl], den1_o.at[sl])

    out_t = (jax.ShapeDtypeStruct((den_n, 16), jnp.float32),
             jax.ShapeDtypeStruct((den_n, 16), jnp.float32))
    scratch = [
        pltpu.VMEM_SHARED((den_n, 16), jnp.float32),
        pltpu.VMEM((2, _K), jnp.int32),
        pltpu.VMEM((2, _K), jnp.int32),
        pltpu.VMEM((_K,), jnp.int32),
        pltpu.VMEM((2, _K, 16), jnp.float32),
        pltpu.VMEM((2, _K, 16), jnp.float32),
        pltpu.VMEM((_K, 16), jnp.float32),
        pltpu.VMEM((rows_t // 16, 16), jnp.float32),
        pltpu.SemaphoreType.DMA,
        pltpu.SemaphoreType.DMA,
    ]
    args = []
    for k in keys:
        args.extend(edge_pads[k])
    args.extend(as_tabs)
    args.extend(ad_tabs)
    kern = pl.kernel(body, out_type=out_t, mesh=_mesh(), scratch_types=scratch,
                     compiler_params=_SC_PARAMS)
    return kern(*args)


def _alpha_pass(jobs, den_bases, a_bases, a_tot, as_tabs, ad_tabs,
                den0, den1, edge_pads):
    keys = _edge_keys(jobs)
    n_e = 2 * len(keys)
    n_j = len(jobs)

    def body(*refs):
        e_flat = refs[:n_e]
        asr = refs[n_e:n_e + n_j]
        adr = refs[n_e + n_j:n_e + 2 * n_j]
        den0_r, den1_r = refs[n_e + 2 * n_j], refs[n_e + 2 * n_j + 1]
        alpha_o = refs[n_e + 2 * n_j + 2]
        (idx_s, idx_d, idx_p, asb, adb, d0b, d1b, ab, sem0, sem1) = \
            refs[n_e + 2 * n_j + 3:]
        e_refs = {k: (e_flat[2 * i], e_flat[2 * i + 1])
                  for i, k in enumerate(keys)}
        c = lax.axis_index("c")
        s = lax.axis_index("s")
        wid = c * _NTILE + s

        for ji, (r, key, swap) in enumerate(jobs):
            s_ref, d_ref = _job_edges(e_refs, key, swap)
            e_cnt = _E_KEY[key]
            nb = _epad(e_cnt) // _K
            nblk = (nb - wid + _NW - 1) // _NW
            a_tab, d_tab = asr[ji], adr[ji]
            dbase = den_bases[r]
            abase = a_bases[ji]
            sems = (sem0, sem1)

            def fetch(bi, slot, s_ref=s_ref, d_ref=d_ref, a_tab=a_tab,
                      d_tab=d_tab, dbase=dbase, sems=sems):
                base = (wid + bi * _NW) * _K
                pltpu.sync_copy(s_ref.at[pl.ds(base, _K)], idx_s.at[slot])
                pltpu.sync_copy(d_ref.at[pl.ds(base, _K)], idx_d.at[slot])

                def col(m, cc):
                    idx_p[slot, pl.ds(m * 16, 16)] = \
                        idx_d[slot, pl.ds(m * 16, 16)] + dbase
                    return cc
                lax.fori_loop(0, _K // 16, col, None, unroll=4)
                pltpu.async_copy(a_tab.at[idx_s.at[slot]], asb.at[slot],
                                 sems[slot])
                pltpu.async_copy(d_tab.at[idx_d.at[slot]], adb.at[slot],
                                 sems[slot])
                pltpu.async_copy(den0_r.at[idx_p.at[slot]], d0b.at[slot],
                                 sems[slot])
                pltpu.async_copy(den1_r.at[idx_p.at[slot]], d1b.at[slot],
                                 sems[slot])

            @pl.when(nblk > 0)
            def _(fetch=fetch):
                fetch(0, 0)

            def pair(p, carry, fetch=fetch, a_tab=a_tab, d_tab=d_tab,
                     e_cnt=e_cnt, abase=abase, nblk=nblk, sems=sems):
                for b in (0, 1):
                    bi = 2 * p + b

                    @pl.when(bi < nblk)
                    def _(bi=bi, b=b):
                        pltpu.make_async_copy(a_tab.at[idx_s.at[b]],
                                              asb.at[b], sems[b]).wait()
                        pltpu.make_async_copy(d_tab.at[idx_d.at[b]],
                                              adb.at[b], sems[b]).wait()
                        pltpu.make_async_copy(den0_r.at[idx_p.at[b]],
                                              d0b.at[b], sems[b]).wait()
                        pltpu.make_async_copy(den1_r.at[idx_p.at[b]],
                                              d1b.at[b], sems[b]).wait()

                        @pl.when(bi + 1 < nblk)
                        def _():
                            fetch(bi + 1, 1 - b)
                        base = (wid + bi * _NW) * _K

                        def edge(k, cc):
                            e = asb[b, k] + adb[b, k]
                            e = jnp.where(e > 0, e, 0.2 * e)
                            ex = jnp.exp(jnp.minimum(e, 60.0))
                            valid = jnp.where(base + k < e_cnt, 1.0, 0.0)
                            den = d0b[b, k] + d1b[b, k] + 1e-16
                            ab[k] = ex * valid / den
                            return cc
                        lax.fori_loop(0, _K, edge, None, unroll=4)
                        pltpu.sync_copy(ab, alpha_o.at[pl.ds(abase + base,
                                                             _K)])
                return carry

            lax.fori_loop(0, (nblk + 1) // 2, pair, None)

    out_t = jax.ShapeDtypeStruct((a_tot, 16), jnp.float32)
    scratch = ([pltpu.VMEM((2, _K), jnp.int32)] * 3 +
               [pltpu.VMEM((2, _K, 16), jnp.float32)] * 4 +
               [pltpu.VMEM((_K, 16), jnp.float32),
                pltpu.SemaphoreType.DMA, pltpu.SemaphoreType.DMA])
    args = []
    for k in keys:
        args.extend(edge_pads[k])
    args.extend(as_tabs)
    args.extend(ad_tabs)
    args.extend([den0, den1])
    kern = pl.kernel(body, out_type=out_t, mesh=_mesh(), scratch_types=scratch,
                     compiler_params=_SC_PARAMS)
    return kern(*args)


def _msg_pass(jobs, a_bases, alpha, tabs, tab_of_r, groups, nch, heads,
              edge_pads, biases, pdt):
    keys = _edge_keys(jobs)
    n_e = 2 * len(keys)
    n_x = len(_TYPES)
    n_g = len(groups)
    acc_rows = max(_npad(_NNODE[g[0]]) for g in groups)

    def body(*refs):
        e_flat = refs[:n_e]
        xst = {t: refs[n_e + i] for i, t in enumerate(_TYPES)}
        alpha_r = refs[n_e + n_x]
        b_refs = refs[n_e + n_x + 1:n_e + n_x + 1 + n_g]
        outs = refs[n_e + n_x + 1 + n_g:n_e + n_x + 1 + 2 * n_g]
        (acc, idx_s, idx_d, arows, rows, zb, bbuf, sem0, sem1) = \
            refs[n_e + n_x + 1 + 2 * n_g:]
        e_refs = {k: (e_flat[2 * i], e_flat[2 * i + 1])
                  for i, k in enumerate(keys)}
        c = lax.axis_index("c")
        s = lax.axis_index("s")

        for gi, (dst_t, job_ids) in enumerate(groups):
            npad_d = _npad(_NNODE[dst_t])
            rt = npad_d // _NTILE

            def chunk(cc_l, carry, gi=gi, job_ids=job_ids, rt=rt):
                cc = cc_l * 2 + c
                lane = cc_l if heads > 1 else 0
                pltpu.sync_copy(b_refs[gi].at[pl.ds(cc * 64, 64)], bbuf)
                lanes = 32 if pdt == jnp.bfloat16 else 16
                nq = 64 // lanes

                def brow(m, bc):
                    q = m % nq
                    sl = pl.ds(q * lanes, lanes)
                    zb[m // nq, sl] = bbuf[sl]
                    return bc
                lax.fori_loop(0, 64 * nq, brow, None)
                for q in range(rt // 64):
                    pltpu.sync_copy(zb, acc.at[pl.ds(s * rt + q * 64, 64)])
                plsc.subcore_barrier()

                for ji in job_ids:
                    r, key, swap = jobs[ji]
                    s_ref, d_ref = _job_edges(e_refs, key, swap)
                    nb = _epad(_E_KEY[key]) // _K
                    nblk = (nb - s + _NTILE - 1) // _NTILE
                    t_r, cbase = tab_of_r[r]
                    tab = xst[t_r]
                    tcc = cbase + cc
                    abase = a_bases[ji]
                    sems = (sem0, sem1)

                    def fetch(bi, slot, s_ref=s_ref, d_ref=d_ref, tab=tab,
                              abase=abase, cc=tcc, sems=sems):
                        base = (s + bi * _NTILE) * _K
                        pltpu.sync_copy(s_ref.at[pl.ds(base, _K)],
                                        idx_s.at[slot])
                        pltpu.sync_copy(d_ref.at[pl.ds(base, _K)],
                                        idx_d.at[slot])
                        pltpu.sync_copy(alpha_r.at[pl.ds(abase + base, _K)],
                                        arows.at[slot])
                        pltpu.async_copy(tab.at[cc].at[idx_s.at[slot]],
                                         rows.at[slot], sems[slot])

                    @pl.when(nblk > 0)
                    def _(fetch=fetch):
                        fetch(0, 0)

                    def pair(p, bc, fetch=fetch, tab=tab, cc=tcc, lane=lane,
                             nblk=nblk, sems=sems):
                        ll = jnp.full((16,), lane, jnp.int32)
                        for b in (0, 1):
                            bi = 2 * p + b

                            @pl.when(bi < nblk)
                            def _(bi=bi, b=b):
                                pltpu.make_async_copy(
                                    tab.at[cc].at[idx_s.at[b]],
                                    rows.at[b], sems[b]).wait()

                                @pl.when(bi + 1 < nblk)
                                def _():
                                    fetch(bi + 1, 1 - b)

                                def edge(k, ec):
                                    kk = jnp.full((16,), k, jnp.int32)
                                    av = plsc.load_gather(arows.at[b],
                                                          [kk, ll])
                                    if pdt == jnp.bfloat16:
                                        for q in range(2):
                                            sl = pl.ds(q * 32, 32)
                                            v = rows[b, k, sl]
                                            lo, hi = plsc.unpack(
                                                v, format=plsc.PackFormat
                                                .INTERLEAVED)
                                            rows[b, k, sl] = plsc.pack(
                                                lo * av, hi * av,
                                                format=plsc.PackFormat
                                                .INTERLEAVED)
                                    else:
                                        for q in range(4):
                                            sl = pl.ds(q * 16, 16)
                                            rows[b, k, sl] = \
                                                rows[b, k, sl] * av
                                    return ec
                                lax.fori_loop(0, _K, edge, None, unroll=4)
                                pltpu.sync_copy(rows.at[b],
                                                acc.at[idx_d.at[b]], add=True)
                        return bc

                    lax.fori_loop(0, (nblk + 1) // 2, pair, None)

                plsc.subcore_barrier()
                sl = pl.ds(s * rt, rt)
                pltpu.sync_copy(acc.at[sl], outs[gi].at[cc].at[sl])
                plsc.subcore_barrier()
                return carry

            lax.fori_loop(0, nch // 2, chunk, None)

    out_t = tuple(jax.ShapeDtypeStruct((nch, _npad(_NNODE[g[0]]), 64),
                                       pdt) for g in groups)
    scratch = [
        pltpu.VMEM_SHARED((acc_rows, 64), pdt),
        pltpu.VMEM((2, _K), jnp.int32),
        pltpu.VMEM((2, _K), jnp.int32),
        pltpu.VMEM((2, _K, 16), jnp.float32),
        pltpu.VMEM((2, _K, 64), pdt),
        pltpu.VMEM((64, 64), pdt),
        pltpu.VMEM((64,), pdt),
        pltpu.SemaphoreType.DMA,
        pltpu.SemaphoreType.DMA,
    ]
    args = []
    for k in keys:
        args.extend(edge_pads[k])
    args.extend(tabs[t] for t in _TYPES)
    args.append(alpha)
    args.extend(biases)
    kern = pl.kernel(body, out_type=out_t, mesh=_mesh(), scratch_types=scratch,
                     compiler_params=_SC_PARAMS)
    return kern(*args)


_KF = 32


def _edge_dot(eli_u, eli_p, hu, hp, e_pad):
    d = hu.shape[1]

    def body(u_ref, p_ref, hu_ref, hp_ref, out_ref,
             idx_u, idx_p, urows, prows, resb, sem0, sem1):
        c = lax.axis_index("c")
        s = lax.axis_index("s")
        wid = c * _NTILE + s
        nb = e_pad // _KF
        nblk = (nb - wid + _NW - 1) // _NW
        sems = (sem0, sem1)

        def fetch(bi, slot):
            base = (wid + bi * _NW) * _KF
            pltpu.sync_copy(u_ref.at[pl.ds(base, _KF)], idx_u.at[slot])
            pltpu.sync_copy(p_ref.at[pl.ds(base, _KF)], idx_p.at[slot])
            pltpu.async_copy(hu_ref.at[idx_u.at[slot]], urows.at[slot],
                             sems[slot])
            pltpu.async_copy(hp_ref.at[idx_p.at[slot]], prows.at[slot],
                             sems[slot])

        @pl.when(nblk > 0)
        def _():
            fetch(0, 0)

        iot = lax.iota(jnp.int32, 16)

        def pair(p, carry):
            for b in (0, 1):
                bi = 2 * p + b

                @pl.when(bi < nblk)
                def _(bi=bi, b=b):
                    pltpu.make_async_copy(hu_ref.at[idx_u.at[b]],
                                          urows.at[b], sems[b]).wait()
                    pltpu.make_async_copy(hp_ref.at[idx_p.at[b]],
                                          prows.at[b], sems[b]).wait()

                    @pl.when(bi + 1 < nblk)
                    def _():
                        fetch(bi + 1, 1 - b)
                    base = (wid + bi * _NW) * _KF

                    def egrp(g, ec):
                        kk = iot + g * 16

                        def colj(j, acc):
                            jj = jnp.full((16,), j, jnp.int32)
                            uj = plsc.load_gather(urows.at[b], [kk, jj])
                            pj = plsc.load_gather(prows.at[b], [kk, jj])
                            return acc + uj * pj
                        res = lax.fori_loop(0, d, colj,
                                            jnp.zeros((16,), jnp.float32),
                                            unroll=8)
                        resb[pl.ds(g * 16, 16)] = res
                        return ec
                    lax.fori_loop(0, _KF // 16, egrp, None)
                    pltpu.sync_copy(resb, out_ref.at[pl.ds(base, _KF)])
            return carry

        lax.fori_loop(0, (nblk + 1) // 2, pair, None)

    out_t = jax.ShapeDtypeStruct((e_pad,), jnp.float32)
    scratch = [
        pltpu.VMEM((2, _KF), jnp.int32),
        pltpu.VMEM((2, _KF), jnp.int32),
        pltpu.VMEM((2, _KF, d), jnp.float32),
        pltpu.VMEM((2, _KF, d), jnp.float32),
        pltpu.VMEM((_KF,), jnp.float32),
        pltpu.SemaphoreType.DMA,
        pltpu.SemaphoreType.DMA,
    ]
    kern = pl.kernel(body, out_type=out_t, mesh=_mesh(), scratch_types=scratch,
                     compiler_params=_SC_PARAMS)
    return kern(eli_u, eli_p, hu, hp)


def _to_chunks(y):
    n, w = y.shape
    return y.reshape(n, w // 64, 64).transpose(1, 0, 2)


def _from_chunks(yc, n):
    return yc.transpose(1, 0, 2).reshape(yc.shape[1], -1)[:n]


def _pad1(a, n):
    a = a.astype(jnp.int32)
    return jnp.concatenate([a, jnp.zeros((n - a.shape[0],), jnp.int32)])


def _layer(xs_by_type, jobs, rels, big, w_cat, wf, relu, cin, edge_pads, nch,
           heads, groups, biases, pdt):
    den_bases, den_n = _den_layout(rels)
    a_bases, a_tot = _alpha_layout(jobs)

    yb = {t: _matmul(xs_by_type[t], w_cat[t], relu=relu, low=True,
                     cin=cin, cout=True, odt=pdt) for t in _TYPES}
    ya = {t: _matmul(xs_by_type[t], wf, relu=relu, cin=cin) for t in _TYPES}

    tab_of_r, as_by_r, ad_by_r = {}, {}, {}
    for t in _TYPES:
        for i, r in enumerate(big[t]):
            tab_of_r[r] = (t, i * nch)
        for r in rels:
            if _R_SRC[r] == t:
                as_by_r[r] = ya[t][:, r * 32: r * 32 + 16]
            if _R_DST[r] == t:
                ad_by_r[r] = ya[t][:, r * 32 + 16: r * 32 + 32]

    as_list = tuple(as_by_r[r] for r, _k, _s in jobs)
    ad_list = tuple(ad_by_r[r] for r, _k, _s in jobs)
    den0, den1 = _den_pass(jobs, den_bases, den_n, as_list, ad_list, edge_pads)
    alpha = _alpha_pass(jobs, den_bases, a_bases, a_tot, as_list, ad_list,
                        den0, den1, edge_pads)
    outs = _msg_pass(jobs, a_bases, alpha, yb, tab_of_r, groups, nch, heads,
                     edge_pads, [b.astype(pdt) for b in biases], pdt)
    return {g[0]: o for g, o in zip(groups, outs)}


def kernel(x_user, x_paper, x_method, x_task, ei_cites, ei_applies,
           ei_performs, ei_likes, edge_label_index, W1, as1, ad1, b1,
           W2, as2, ad2, b2):
    xs = {"user": x_user, "paper": x_paper, "method": x_method,
          "task": x_task}

    edge_pads = {}
    for key, ei in (("cites", ei_cites), ("applies", ei_applies),
                    ("performs", ei_performs), ("likes", ei_likes)):
        ep = _epad(_E_KEY[key])
        edge_pads[key] = (_pad1(ei[0], ep), _pad1(ei[1], ep))

    wf1 = _fold(W1, as1, ad1, _H1, _C1)
    wf2 = _fold(W2, as2, ad2, 1, _D)
    wf1 = jnp.concatenate([wf1, jnp.zeros((_D, 32), jnp.float32)], axis=1)
    wf2 = jnp.concatenate([wf2, jnp.zeros((_DH, 32), jnp.float32)], axis=1)

    m_dst = jnp.array([[1.0 if _R_DST[r] == t else 0.0 for r in range(7)]
                       for t in _TYPES], jnp.float32)
    b1s = _matmul(m_dst, b1, bm=8)
    b2s = _matmul(m_dst, b2, bm=8)

    big1 = {"paper": [0, 1, 3, 6], "method": [2], "task": [4], "user": [5]}
    w_cat1 = {t: jnp.concatenate([W1[r] for r in big1[t]], axis=1)
              for t in _TYPES}
    groups1 = [("paper", [0, 1, 3, 5, 6]), ("method", [2]), ("task", [4]),
               ("user", [7])]
    bias_g1 = [b1s[1], b1s[2], b1s[3], b1s[0]]
    outs1 = _layer(xs, _L1_JOBS, _L1_RELS, big1, w_cat1, wf1, False, False,
                   edge_pads, 16, _H1, groups1, bias_g1, jnp.bfloat16)

    big2 = {"paper": [0, 6], "method": [2], "task": [4], "user": [5]}
    w_cat2 = {t: jnp.concatenate([W2[r] for r in big2[t]], axis=1)
              for t in _TYPES}
    groups2 = [("paper", [0, 1, 2, 3, 4]), ("user", [5])]
    bias_g2 = [b2s[1], b2s[0]]
    outs2 = _layer(outs1, _L2_JOBS, _L2_RELS, big2, w_cat2, wf2, True, True,
                   edge_pads, 12, 1, groups2, bias_g2, jnp.float32)
    h2u = _from_chunks(outs2["user"], _NU).astype(jnp.float32)
    h2p = _from_chunks(outs2["paper"], _NP).astype(jnp.float32)

    el = edge_label_index.shape[1]
    el_pad = _cdiv(el, _KF * _NW) * _KF * _NW
    eli_u = _pad1(edge_label_index[0], el_pad)
    eli_p = _pad1(edge_label_index[1], el_pad)
    res = _edge_dot(eli_u, eli_p, h2u, h2p, el_pad)
    return res[:el]


# speedup vs baseline: 1.1611x; 1.1408x over previous
import functools

import jax
import jax.numpy as jnp
from jax import lax
from jax.experimental import pallas as pl
from jax.experimental.pallas import tpu as pltpu
from jax.experimental.pallas import tpu_sc as plsc

_NU, _NP, _NM, _NT = 5000, 20000, 2000, 1000
_D, _DH = 768, 1024
_H1, _C1 = 8, 128
_K = 256
_NSC, _NTILE = 2, 16
_NW = _NSC * _NTILE

_TYPES = ("user", "paper", "method", "task")
_NNODE = {"user": _NU, "paper": _NP, "method": _NM, "task": _NT}
_R_SRC = ("paper", "paper", "method", "paper", "task", "user", "paper")
_R_DST = ("paper", "method", "paper", "task", "paper", "paper", "user")
_E_KEY = {"cites": 20000, "applies": 10000, "performs": 10000, "likes": 12000}

_L1_JOBS = [(0, "cites", False), (0, "cites", True), (1, "applies", False),
            (2, "applies", True), (3, "performs", False), (4, "performs", True),
            (5, "likes", False), (6, "likes", True)]
_L2_JOBS = [(0, "cites", False), (0, "cites", True), (2, "applies", True),
            (4, "performs", True), (5, "likes", False), (6, "likes", True)]

_L1_RELS = (0, 1, 2, 3, 4, 5, 6)
_L2_RELS = (0, 2, 4, 5, 6)


def _cdiv(a, b):
    return (a + b - 1) // b


def _epad(e):
    return _cdiv(e, _K) * _K


def _npad(n):
    return _cdiv(n, 1024) * 1024


def _den_layout(rels):
    bases, off = {}, 0
    for r in rels:
        bases[r] = off
        off += _NNODE[_R_DST[r]]
    off = _cdiv(off, 2048) * 2048
    return bases, off


def _alpha_layout(jobs):
    bases, off = [], 0
    for (_r, key, _s) in jobs:
        bases.append(off)
        off += _epad(_E_KEY[key])
    return bases, off


def _mm_body(relu, low, cin, cout, cb, cw, odt, x_ref, w_ref, o_ref):
    a = x_ref[...]
    if cin:
        a = a.transpose(1, 0, 2).reshape(a.shape[1],
                                         a.shape[0] * a.shape[2])
    if relu:
        a = jnp.maximum(a, 0.0)
    b = w_ref[...]
    if low:
        a = a.astype(jnp.bfloat16)
        b = b.astype(jnp.bfloat16)
    else:
        a = a.astype(jnp.float32)
    res = jnp.dot(a, b, preferred_element_type=jnp.float32)
    res = res.astype(odt)
    if cout:
        res = res.reshape(res.shape[0], cb, cw).transpose(1, 0, 2)
    o_ref[...] = res


def _matmul(x, w, relu=False, low=False, cin=False, cout=False, bm=256,
            odt=jnp.float32, cw=64):
    if cin:
        nch_in, m, w_in = x.shape
        k = nch_in * w_in
    else:
        m, k = x.shape
    n = w.shape[1]
    bn = 512 if n % 512 == 0 else 256
    cb = bn // cw
    grid = (_cdiv(m, bm), _cdiv(n, bn))
    if cin:
        x_spec = pl.BlockSpec((nch_in, bm, w_in), lambda i, j: (0, i, 0))
    else:
        x_spec = pl.BlockSpec((bm, k), lambda i, j: (i, 0))
    if cout:
        o_spec = pl.BlockSpec((cb, bm, cw), lambda i, j: (j, i, 0))
        o_shape = jax.ShapeDtypeStruct((n // cw, m, cw), odt)
    else:
        o_spec = pl.BlockSpec((bm, bn), lambda i, j: (i, j))
        o_shape = jax.ShapeDtypeStruct((m, n), odt)
    return pl.pallas_call(
        functools.partial(_mm_body, relu, low, cin, cout, cb, cw, odt),
        grid=grid,
        in_specs=[x_spec, pl.BlockSpec((k, bn), lambda i, j: (0, j))],
        out_specs=o_spec,
        out_shape=o_shape,
    )(x, w)


def _fold_body(h, c, w_ref, as_ref, ad_ref, o_ref):
    d = w_ref.shape[1]
    w = w_ref[0].reshape(d, h, c)
    was = (w * as_ref[0][None]).sum(-1)
    wad = (w * ad_ref[0][None]).sum(-1)
    z = jnp.zeros((d, 16 - h), jnp.float32)
    o_ref[0] = jnp.concatenate([was, z, wad, z], axis=1)


def _fold(w, a_s, a_d, h, c):
    r, d, _ = w.shape
    out = pl.pallas_call(
        functools.partial(_fold_body, h, c),
        grid=(r,),
        in_specs=[pl.BlockSpec((1, d, h * c), lambda i: (i, 0, 0)),
                  pl.BlockSpec((1, h, c), lambda i: (i, 0, 0)),
                  pl.BlockSpec((1, h, c), lambda i: (i, 0, 0))],
        out_specs=pl.BlockSpec((1, d, 32), lambda i: (i, 0, 0)),
        out_shape=jax.ShapeDtypeStruct((r, d, 32), jnp.float32),
    )(w, a_s, a_d)
    return out.transpose(1, 0, 2).reshape(d, r * 32)


_SC_PARAMS = pltpu.CompilerParams(use_tc_tiling_on_sc=False,
                                  needs_layout_passes=False)


def _mesh():
    return plsc.VectorSubcoreMesh(core_axis_name="c", subcore_axis_name="s",
                                  num_cores=_NSC, num_subcores=_NTILE)


def _job_edges(e_refs, key, swap):
    s_ref, d_ref = e_refs[key]
    return (d_ref, s_ref) if swap else (s_ref, d_ref)


def _edge_keys(jobs):
    seen = []
    for (_r, key, _s) in jobs:
        if key not in seen:
            seen.append(key)
    return seen


def _den_pass(jobs, den_bases, den_n, as_tabs, ad_tabs, edge_pads):
    keys = _edge_keys(jobs)
    n_e = 2 * len(keys)
    n_j = len(jobs)
    rows_t = den_n // _NTILE

    def body(*refs):
        e_flat = refs[:n_e]
        asr = refs[n_e:n_e + n_j]
        adr = refs[n_e + n_j:n_e + 2 * n_j]
        den0_o, den1_o = refs[n_e + 2 * n_j], refs[n_e + 2 * n_j + 1]
        (acc, idx_s, idx_d, idx_p, asb, adb, exb, zb, sem0, sem1) = \
            refs[n_e + 2 * n_j + 2:]
        e_refs = {k: (e_flat[2 * i], e_flat[2 * i + 1])
                  for i, k in enumerate(keys)}
        c = lax.axis_index("c")
        s = lax.axis_index("s")
        wid = c * _NTILE + s

        zbr = rows_t // 16

        def zrow(m, carry):
            zb[m] = jnp.zeros((16,), jnp.float32)
            return carry
        lax.fori_loop(0, zbr, zrow, None)
        for q in range(16):
            pltpu.sync_copy(zb, acc.at[pl.ds(s * rows_t + q * zbr, zbr)])
        plsc.subcore_barrier()

        for ji, (r, key, swap) in enumerate(jobs):
            s_ref, d_ref = _job_edges(e_refs, key, swap)
            e_cnt = _E_KEY[key]
            nb = _epad(e_cnt) // _K
            nblk = (nb - wid + _NW - 1) // _NW
            a_tab, d_tab = asr[ji], adr[ji]
            dbase = den_bases[r]
            sems = (sem0, sem1)

            def fetch(bi, slot, s_ref=s_ref, d_ref=d_ref, a_tab=a_tab,
                      d_tab=d_tab, sems=sems):
                base = (wid + bi * _NW) * _K
                pltpu.sync_copy(s_ref.at[pl.ds(base, _K)], idx_s.at[slot])
                pltpu.sync_copy(d_ref.at[pl.ds(base, _K)], idx_d.at[slot])
                pltpu.async_copy(a_tab.at[idx_s.at[slot]], asb.at[slot],
                                 sems[slot])
                pltpu.async_copy(d_tab.at[idx_d.at[slot]], adb.at[slot],
                                 sems[slot])

            @pl.when(nblk > 0)
            def _(fetch=fetch):
                fetch(0, 0)

            def pair(p, carry, fetch=fetch, a_tab=a_tab, d_tab=d_tab,
                     dbase=dbase, e_cnt=e_cnt, nblk=nblk, sems=sems):
                for b in (0, 1):
                    bi = 2 * p + b

                    @pl.when(bi < nblk)
                    def _(bi=bi, b=b):
                        pltpu.make_async_copy(a_tab.at[idx_s.at[b]],
                                              asb.at[b], sems[b]).wait()
                        pltpu.make_async_copy(d_tab.at[idx_d.at[b]],
                                              adb.at[b], sems[b]).wait()

                        @pl.when(bi + 1 < nblk)
                        def _():
                            fetch(bi + 1, 1 - b)
                        base = (wid + bi * _NW) * _K

                        def col(m, cc):
                            idx_p[pl.ds(m * 16, 16)] = \
                                idx_d[b, pl.ds(m * 16, 16)] + dbase
                            return cc
                        lax.fori_loop(0, _K // 16, col, None, unroll=4)

                        def edge(k, cc):
                            e = asb[b, k] + adb[b, k]
                            e = jnp.where(e > 0, e, 0.2 * e)
                            ex = jnp.exp(jnp.minimum(e, 60.0))
                            valid = jnp.where(base + k < e_cnt, 1.0, 0.0)
                            exb[k] = ex * valid
                            return cc
                        lax.fori_loop(0, _K, edge, None, unroll=4)
                        pltpu.sync_copy(exb, acc.at[idx_p], add=True)
                return carry

            lax.fori_loop(0, (nblk + 1) // 2, pair, None)

        plsc.subcore_barrier()
        sl = pl.ds(s * rows_t, rows_t)

        @pl.when(c == 0)
        def _():
            pltpu.sync_copy(acc.at[sl], den0_o.at[sl])

        @pl.when(c == 1)
        def _():
            pltpu.sync_copy(acc.at[sl], den1_o.at[sl])

    out_t = (jax.ShapeDtypeStruct((den_n, 16), jnp.float32),
             jax.ShapeDtypeStruct((den_n, 16), jnp.float32))
    scratch = [
        pltpu.VMEM_SHARED((den_n, 16), jnp.float32),
        pltpu.VMEM((2, _K), jnp.int32),
        pltpu.VMEM((2, _K), jnp.int32),
        pltpu.VMEM((_K,), jnp.int32),
        pltpu.VMEM((2, _K, 16), jnp.float32),
        pltpu.VMEM((2, _K, 16), jnp.float32),
        pltpu.VMEM((_K, 16), jnp.float32),
        pltpu.VMEM((rows_t // 16, 16), jnp.float32),
        pltpu.SemaphoreType.DMA,
        pltpu.SemaphoreType.DMA,
    ]
    args = []
    for k in keys:
        args.extend(edge_pads[k])
    args.extend(as_tabs)
    args.extend(ad_tabs)
    kern = pl.kernel(body, out_type=out_t, mesh=_mesh(), scratch_types=scratch,
                     compiler_params=_SC_PARAMS)
    return kern(*args)


def _alpha_pass(jobs, den_bases, a_bases, a_tot, as_tabs, ad_tabs,
                den0, den1, edge_pads):
    keys = _edge_keys(jobs)
    n_e = 2 * len(keys)
    n_j = len(jobs)

    def body(*refs):
        e_flat = refs[:n_e]
        asr = refs[n_e:n_e + n_j]
        adr = refs[n_e + n_j:n_e + 2 * n_j]
        den0_r, den1_r = refs[n_e + 2 * n_j], refs[n_e + 2 * n_j + 1]
        alpha_o = refs[n_e + 2 * n_j + 2]
        (idx_s, idx_d, idx_p, asb, adb, d0b, d1b, ab, sem0, sem1) = \
            refs[n_e + 2 * n_j + 3:]
        e_refs = {k: (e_flat[2 * i], e_flat[2 * i + 1])
                  for i, k in enumerate(keys)}
        c = lax.axis_index("c")
        s = lax.axis_index("s")
        wid = c * _NTILE + s

        for ji, (r, key, swap) in enumerate(jobs):
            s_ref, d_ref = _job_edges(e_refs, key, swap)
            e_cnt = _E_KEY[key]
            nb = _epad(e_cnt) // _K
            nblk = (nb - wid + _NW - 1) // _NW
            a_tab, d_tab = asr[ji], adr[ji]
            dbase = den_bases[r]
            abase = a_bases[ji]
            sems = (sem0, sem1)

            def fetch(bi, slot, s_ref=s_ref, d_ref=d_ref, a_tab=a_tab,
                      d_tab=d_tab, dbase=dbase, sems=sems):
                base = (wid + bi * _NW) * _K
                pltpu.sync_copy(s_ref.at[pl.ds(base, _K)], idx_s.at[slot])
                pltpu.sync_copy(d_ref.at[pl.ds(base, _K)], idx_d.at[slot])

                def col(m, cc):
                    idx_p[slot, pl.ds(m * 16, 16)] = \
                        idx_d[slot, pl.ds(m * 16, 16)] + dbase
                    return cc
                lax.fori_loop(0, _K // 16, col, None, unroll=4)
                pltpu.async_copy(a_tab.at[idx_s.at[slot]], asb.at[slot],
                                 sems[slot])
                pltpu.async_copy(d_tab.at[idx_d.at[slot]], adb.at[slot],
                                 sems[slot])
                pltpu.async_copy(den0_r.at[idx_p.at[slot]], d0b.at[slot],
                                 sems[slot])
                pltpu.async_copy(den1_r.at[idx_p.at[slot]], d1b.at[slot],
                                 sems[slot])

            @pl.when(nblk > 0)
            def _(fetch=fetch):
                fetch(0, 0)

            def pair(p, carry, fetch=fetch, a_tab=a_tab, d_tab=d_tab,
                     e_cnt=e_cnt, abase=abase, nblk=nblk, sems=sems):
                for b in (0, 1):
                    bi = 2 * p + b

                    @pl.when(bi < nblk)
                    def _(bi=bi, b=b):
                        pltpu.make_async_copy(a_tab.at[idx_s.at[b]],
                                              asb.at[b], sems[b]).wait()
                        pltpu.make_async_copy(d_tab.at[idx_d.at[b]],
                                              adb.at[b], sems[b]).wait()
                        pltpu.make_async_copy(den0_r.at[idx_p.at[b]],
                                              d0b.at[b], sems[b]).wait()
                        pltpu.make_async_copy(den1_r.at[idx_p.at[b]],
                                              d1b.at[b], sems[b]).wait()

                        @pl.when(bi + 1 < nblk)
                        def _():
                            fetch(bi + 1, 1 - b)
                        base = (wid + bi * _NW) * _K

                        def edge(k, cc):
                            e = asb[b, k] + adb[b, k]
                            e = jnp.where(e > 0, e, 0.2 * e)
                            ex = jnp.exp(jnp.minimum(e, 60.0))
                            valid = jnp.where(base + k < e_cnt, 1.0, 0.0)
                            den = d0b[b, k] + d1b[b, k] + 1e-16
                            ab[k] = ex * valid / den
                            return cc
                        lax.fori_loop(0, _K, edge, None, unroll=4)
                        pltpu.sync_copy(ab, alpha_o.at[pl.ds(abase + base,
                                                             _K)])
                return carry

            lax.fori_loop(0, (nblk + 1) // 2, pair, None)

    out_t = jax.ShapeDtypeStruct((a_tot, 16), jnp.float32)
    scratch = ([pltpu.VMEM((2, _K), jnp.int32)] * 3 +
               [pltpu.VMEM((2, _K, 16), jnp.float32)] * 4 +
               [pltpu.VMEM((_K, 16), jnp.float32),
                pltpu.SemaphoreType.DMA, pltpu.SemaphoreType.DMA])
    args = []
    for k in keys:
        args.extend(edge_pads[k])
    args.extend(as_tabs)
    args.extend(ad_tabs)
    args.extend([den0, den1])
    kern = pl.kernel(body, out_type=out_t, mesh=_mesh(), scratch_types=scratch,
                     compiler_params=_SC_PARAMS)
    return kern(*args)


def _msg_pass(jobs, a_bases, alpha, tabs, tab_of_r, groups, nch, heads,
              edge_pads, biases, pdt, cw):
    keys = _edge_keys(jobs)
    n_e = 2 * len(keys)
    n_x = len(_TYPES)
    n_g = len(groups)
    acc_rows = max(_npad(_NNODE[g[0]]) for g in groups)

    def body(*refs):
        e_flat = refs[:n_e]
        xst = {t: refs[n_e + i] for i, t in enumerate(_TYPES)}
        alpha_r = refs[n_e + n_x]
        b_refs = refs[n_e + n_x + 1:n_e + n_x + 1 + n_g]
        outs = refs[n_e + n_x + 1 + n_g:n_e + n_x + 1 + 2 * n_g]
        (acc, idx_s, idx_d, arows, rows, zb, bbuf, sem0, sem1) = \
            refs[n_e + n_x + 1 + 2 * n_g:]
        e_refs = {k: (e_flat[2 * i], e_flat[2 * i + 1])
                  for i, k in enumerate(keys)}
        c = lax.axis_index("c")
        s = lax.axis_index("s")

        for gi, (dst_t, job_ids) in enumerate(groups):
            npad_d = _npad(_NNODE[dst_t])
            rt = npad_d // _NTILE

            def chunk(cc_l, carry, gi=gi, job_ids=job_ids, rt=rt):
                cc = cc_l * 2 + c
                lane = (cc * cw) // _C1 if heads > 1 else 0
                pltpu.sync_copy(b_refs[gi].at[pl.ds(cc * cw, cw)], bbuf)
                lanes = 32 if pdt == jnp.bfloat16 else 16
                nq = cw // lanes

                def brow(m, bc):
                    q = m % nq
                    sl = pl.ds(q * lanes, lanes)
                    zb[m // nq, sl] = bbuf[sl]
                    return bc
                lax.fori_loop(0, 64 * nq, brow, None)
                for q in range(rt // 64):
                    pltpu.sync_copy(zb, acc.at[pl.ds(s * rt + q * 64, 64)])
                plsc.subcore_barrier()

                for ji in job_ids:
                    r, key, swap = jobs[ji]
                    s_ref, d_ref = _job_edges(e_refs, key, swap)
                    nb = _epad(_E_KEY[key]) // _K
                    nblk = (nb - s + _NTILE - 1) // _NTILE
                    t_r, cbase = tab_of_r[r]
                    tab = xst[t_r]
                    tcc = cbase + cc
                    abase = a_bases[ji]
                    sems = (sem0, sem1)

                    def fetch(bi, slot, s_ref=s_ref, d_ref=d_ref, tab=tab,
                              abase=abase, cc=tcc, sems=sems):
                        base = (s + bi * _NTILE) * _K
                        pltpu.sync_copy(s_ref.at[pl.ds(base, _K)],
                                        idx_s.at[slot])
                        pltpu.sync_copy(d_ref.at[pl.ds(base, _K)],
                                        idx_d.at[slot])
                        pltpu.sync_copy(alpha_r.at[pl.ds(abase + base, _K)],
                                        arows.at[slot])
                        pltpu.async_copy(tab.at[cc].at[idx_s.at[slot]],
                                         rows.at[slot], sems[slot])

                    @pl.when(nblk > 0)
                    def _(fetch=fetch):
                        fetch(0, 0)

                    def pair(p, bc, fetch=fetch, tab=tab, cc=tcc, lane=lane,
                             nblk=nblk, sems=sems):
                        ll = jnp.full((16,), lane, jnp.int32)
                        for b in (0, 1):
                            bi = 2 * p + b

                            @pl.when(bi < nblk)
                            def _(bi=bi, b=b):
                                pltpu.make_async_copy(
                                    tab.at[cc].at[idx_s.at[b]],
                                    rows.at[b], sems[b]).wait()

                                @pl.when(bi + 1 < nblk)
                                def _():
                                    fetch(bi + 1, 1 - b)

                                def edge(k, ec):
                                    kk = jnp.full((16,), k, jnp.int32)
                                    av = plsc.load_gather(arows.at[b],
                                                          [kk, ll])
                                    if pdt == jnp.bfloat16:
                                        for q in range(cw // 32):
                                            sl = pl.ds(q * 32, 32)
                                            v = rows[b, k, sl]
                                            lo, hi = plsc.unpack(
                                                v, format=plsc.PackFormat
                                                .INTERLEAVED)
                                            rows[b, k, sl] = plsc.pack(
                                                lo * av, hi * av,
                                                format=plsc.PackFormat
                                                .INTERLEAVED)
                                    else:
                                        for q in range(cw // 16):
                                            sl = pl.ds(q * 16, 16)
                                            rows[b, k, sl] = \
                                                rows[b, k, sl] * av
                                    return ec
                                lax.fori_loop(0, _K, edge, None, unroll=4)
                                pltpu.sync_copy(rows.at[b],
                                                acc.at[idx_d.at[b]], add=True)
                        return bc

                    lax.fori_loop(0, (nblk + 1) // 2, pair, None)

                plsc.subcore_barrier()
                sl = pl.ds(s * rt, rt)
                pltpu.sync_copy(acc.at[sl], outs[gi].at[cc].at[sl])
                plsc.subcore_barrier()
                return carry

            lax.fori_loop(0, nch // 2, chunk, None)

    out_t = tuple(jax.ShapeDtypeStruct((nch, _npad(_NNODE[g[0]]), cw),
                                       pdt) for g in groups)
    scratch = [
        pltpu.VMEM_SHARED((acc_rows, cw), pdt),
        pltpu.VMEM((2, _K), jnp.int32),
        pltpu.VMEM((2, _K), jnp.int32),
        pltpu.VMEM((2, _K, 16), jnp.float32),
        pltpu.VMEM((2, _K, cw), pdt),
        pltpu.VMEM((64, cw), pdt),
        pltpu.VMEM((cw,), pdt),
        pltpu.SemaphoreType.DMA,
        pltpu.SemaphoreType.DMA,
    ]
    args = []
    for k in keys:
        args.extend(edge_pads[k])
    args.extend(tabs[t] for t in _TYPES)
    args.append(alpha)
    args.extend(biases)
    kern = pl.kernel(body, out_type=out_t, mesh=_mesh(), scratch_types=scratch,
                     compiler_params=_SC_PARAMS)
    return kern(*args)


_KF = 32


def _edge_dot(eli_u, eli_p, hu, hp, e_pad):
    d = hu.shape[1]

    def body(u_ref, p_ref, hu_ref, hp_ref, out_ref,
             idx_u, idx_p, urows, prows, resb, sem0, sem1):
        c = lax.axis_index("c")
        s = lax.axis_index("s")
        wid = c * _NTILE + s
        nb = e_pad // _KF
        nblk = (nb - wid + _NW - 1) // _NW
        sems = (sem0, sem1)

        def fetch(bi, slot):
            base = (wid + bi * _NW) * _KF
            pltpu.sync_copy(u_ref.at[pl.ds(base, _KF)], idx_u.at[slot])
            pltpu.sync_copy(p_ref.at[pl.ds(base, _KF)], idx_p.at[slot])
            pltpu.async_copy(hu_ref.at[idx_u.at[slot]], urows.at[slot],
                             sems[slot])
            pltpu.async_copy(hp_ref.at[idx_p.at[slot]], prows.at[slot],
                             sems[slot])

        @pl.when(nblk > 0)
        def _():
            fetch(0, 0)

        iot = lax.iota(jnp.int32, 16)

        def pair(p, carry):
            for b in (0, 1):
                bi = 2 * p + b

                @pl.when(bi < nblk)
                def _(bi=bi, b=b):
                    pltpu.make_async_copy(hu_ref.at[idx_u.at[b]],
                                          urows.at[b], sems[b]).wait()
                    pltpu.make_async_copy(hp_ref.at[idx_p.at[b]],
                                          prows.at[b], sems[b]).wait()

                    @pl.when(bi + 1 < nblk)
                    def _():
                        fetch(bi + 1, 1 - b)
                    base = (wid + bi * _NW) * _KF

                    def egrp(g, ec):
                        kk = iot + g * 16

                        def colj(j, acc):
                            jj = jnp.full((16,), j, jnp.int32)
                            uj = plsc.load_gather(urows.at[b], [kk, jj])
                            pj = plsc.load_gather(prows.at[b], [kk, jj])
                            return acc + uj * pj
                        res = lax.fori_loop(0, d, colj,
                                            jnp.zeros((16,), jnp.float32),
                                            unroll=8)
                        resb[pl.ds(g * 16, 16)] = res
                        return ec
                    lax.fori_loop(0, _KF // 16, egrp, None)
                    pltpu.sync_copy(resb, out_ref.at[pl.ds(base, _KF)])
            return carry

        lax.fori_loop(0, (nblk + 1) // 2, pair, None)

    out_t = jax.ShapeDtypeStruct((e_pad,), jnp.float32)
    scratch = [
        pltpu.VMEM((2, _KF), jnp.int32),
        pltpu.VMEM((2, _KF), jnp.int32),
        pltpu.VMEM((2, _KF, d), jnp.float32),
        pltpu.VMEM((2, _KF, d), jnp.float32),
        pltpu.VMEM((_KF,), jnp.float32),
        pltpu.SemaphoreType.DMA,
        pltpu.SemaphoreType.DMA,
    ]
    kern = pl.kernel(body, out_type=out_t, mesh=_mesh(), scratch_types=scratch,
                     compiler_params=_SC_PARAMS)
    return kern(eli_u, eli_p, hu, hp)


def _to_chunks(y):
    n, w = y.shape
    return y.reshape(n, w // 64, 64).transpose(1, 0, 2)


def _from_chunks(yc, n):
    return yc.transpose(1, 0, 2).reshape(yc.shape[1], -1)[:n]


def _pad1(a, n):
    a = a.astype(jnp.int32)
    return jnp.concatenate([a, jnp.zeros((n - a.shape[0],), jnp.int32)])


def _layer(xs_by_type, jobs, rels, big, w_cat, wf, relu, cin, edge_pads, nch,
           heads, groups, biases, pdt, cw):
    den_bases, den_n = _den_layout(rels)
    a_bases, a_tot = _alpha_layout(jobs)

    yb = {t: _matmul(xs_by_type[t], w_cat[t], relu=relu, low=True,
                     cin=cin, cout=True, odt=pdt, cw=cw) for t in _TYPES}
    ya = {t: _matmul(xs_by_type[t], wf, relu=relu, cin=cin) for t in _TYPES}

    tab_of_r, as_by_r, ad_by_r = {}, {}, {}
    for t in _TYPES:
        for i, r in enumerate(big[t]):
            tab_of_r[r] = (t, i * nch)
        for r in rels:
            if _R_SRC[r] == t:
                as_by_r[r] = ya[t][:, r * 32: r * 32 + 16]
            if _R_DST[r] == t:
                ad_by_r[r] = ya[t][:, r * 32 + 16: r * 32 + 32]

    as_list = tuple(as_by_r[r] for r, _k, _s in jobs)
    ad_list = tuple(ad_by_r[r] for r, _k, _s in jobs)
    den0, den1 = _den_pass(jobs, den_bases, den_n, as_list, ad_list, edge_pads)
    alpha = _alpha_pass(jobs, den_bases, a_bases, a_tot, as_list, ad_list,
                        den0, den1, edge_pads)
    outs = _msg_pass(jobs, a_bases, alpha, yb, tab_of_r, groups, nch, heads,
                     edge_pads, [b.astype(pdt) for b in biases], pdt, cw)
    return {g[0]: o for g, o in zip(groups, outs)}


def kernel(x_user, x_paper, x_method, x_task, ei_cites, ei_applies,
           ei_performs, ei_likes, edge_label_index, W1, as1, ad1, b1,
           W2, as2, ad2, b2):
    xs = {"user": x_user, "paper": x_paper, "method": x_method,
          "task": x_task}

    edge_pads = {}
    for key, ei in (("cites", ei_cites), ("applies", ei_applies),
                    ("performs", ei_performs), ("likes", ei_likes)):
        ep = _epad(_E_KEY[key])
        edge_pads[key] = (_pad1(ei[0], ep), _pad1(ei[1], ep))

    wf1 = _fold(W1, as1, ad1, _H1, _C1)
    wf2 = _fold(W2, as2, ad2, 1, _D)
    wf1 = jnp.concatenate([wf1, jnp.zeros((_D, 32), jnp.float32)], axis=1)
    wf2 = jnp.concatenate([wf2, jnp.zeros((_DH, 32), jnp.float32)], axis=1)

    m_dst = jnp.array([[1.0 if _R_DST[r] == t else 0.0 for r in range(7)]
                       for t in _TYPES], jnp.float32)
    b1s = _matmul(m_dst, b1, bm=8)
    b2s = _matmul(m_dst, b2, bm=8)

    big1 = {"paper": [0, 1, 3, 6], "method": [2], "task": [4], "user": [5]}
    w_cat1 = {t: jnp.concatenate([W1[r] for r in big1[t]], axis=1)
              for t in _TYPES}
    groups1 = [("paper", [0, 1, 3, 5, 6]), ("method", [2]), ("task", [4]),
               ("user", [7])]
    bias_g1 = [b1s[1], b1s[2], b1s[3], b1s[0]]
    outs1 = _layer(xs, _L1_JOBS, _L1_RELS, big1, w_cat1, wf1, False, False,
                   edge_pads, 8, _H1, groups1, bias_g1, jnp.bfloat16, 128)

    big2 = {"paper": [0, 6], "method": [2], "task": [4], "user": [5]}
    w_cat2 = {t: jnp.concatenate([W2[r] for r in big2[t]], axis=1)
              for t in _TYPES}
    groups2 = [("paper", [0, 1, 2, 3, 4]), ("user", [5])]
    bias_g2 = [b2s[1], b2s[0]]
    outs2 = _layer(outs1, _L2_JOBS, _L2_RELS, big2, w_cat2, wf2, True, True,
                   edge_pads, 12, 1, groups2, bias_g2, jnp.float32, 64)
    h2u = _from_chunks(outs2["user"], _NU).astype(jnp.float32)
    h2p = _from_chunks(outs2["paper"], _NP).astype(jnp.float32)

    el = edge_label_index.shape[1]
    el_pad = _cdiv(el, _KF * _NW) * _KF * _NW
    eli_u = _pad1(edge_label_index[0], el_pad)
    eli_p = _pad1(edge_label_index[1], el_pad)
    res = _edge_dot(eli_u, eli_p, h2u, h2p, el_pad)
    return res[:el]
